# baseline (speedup 1.0000x reference)
import functools
import math

import jax
import jax.numpy as jnp
from jax import lax
from jax.experimental import pallas as pl
from jax.experimental.pallas import tpu as pltpu
from jax.experimental.pallas import tpu_sc as plsc

N_CONV = 3
FEAT = 128
N_RBF = 20
CUTOFF = 5.0

NQ = 4
QF = FEAT // NQ
NTILES = 16
NCORES = 2

_N = 10000
_E2 = 640000
EPT = _E2 // NTILES
KBLK = 32
NBLK = EPT // KBLK
NGRP = KBLK // 16
_NPAD = 10240
RPT = _NPAD // NTILES

LUTN = 16384
LUTR = 16640
LSCALE = LUTN / CUTOFF
LSTEP = CUTOFF / LUTN


def _phi_body(h_ref, w1_ref, b1_ref, w2_ref, b2_ref, o0, o1, o2, o3):
    h = h_ref[...]
    z = jnp.dot(h, w1_ref[...], preferred_element_type=jnp.float32) + b1_ref[...]
    z = z * jax.nn.sigmoid(z)
    p = jnp.dot(z, w2_ref[...], preferred_element_type=jnp.float32) + b2_ref[...]
    pad = p[:, 0:32]
    o0[...] = jnp.concatenate([p[:, 0:96], pad], axis=1)
    o1[...] = jnp.concatenate([p[:, 96:192], pad], axis=1)
    o2[...] = jnp.concatenate([p[:, 192:288], pad], axis=1)
    o3[...] = jnp.concatenate([p[:, 288:384], pad], axis=1)


def _phi(h, W1, b1, W2p, b2p, block=1000):
    n = h.shape[0]
    oshape = [jax.ShapeDtypeStruct((n, 128), jnp.float32) for _ in range(NQ)]
    return pl.pallas_call(
        _phi_body,
        grid=(n // block,),
        in_specs=[
            pl.BlockSpec((block, FEAT), lambda i: (i, 0)),
            pl.BlockSpec((FEAT, FEAT), lambda i: (0, 0)),
            pl.BlockSpec((1, FEAT), lambda i: (0, 0)),
            pl.BlockSpec((FEAT, 3 * FEAT), lambda i: (0, 0)),
            pl.BlockSpec((1, 3 * FEAT), lambda i: (0, 0)),
        ],
        out_specs=[pl.BlockSpec((block, 128), lambda i: (i, 0)) for _ in range(NQ)],
        out_shape=oshape,
    )(h, W1, b1.reshape(1, -1), W2p, b2p.reshape(1, -1))


def _prep_body(rx, ry, rz, ux_o, uy_o, uz_o, kk_o):
    d = jnp.sqrt(rx[...] ** 2 + ry[...] ** 2 + rz[...] ** 2 + 1e-15)
    ux_o[...] = rx[...] / d
    uy_o[...] = ry[...] / d
    uz_o[...] = rz[...] / d
    kk_o[...] = jnp.minimum(d * LSCALE + 0.5, float(LUTR - 1)).astype(jnp.int32)


def _prep(rx, ry, rz, block=1000):
    rows = rx.shape[0]
    oshape = [jax.ShapeDtypeStruct((rows, 128), jnp.float32) for _ in range(3)]
    oshape.append(jax.ShapeDtypeStruct((rows, 128), jnp.int32))
    spec = pl.BlockSpec((block, 128), lambda i: (i, 0))
    return pl.pallas_call(
        _prep_body,
        grid=(rows // block,),
        in_specs=[spec] * 3,
        out_specs=[spec] * 4,
        out_shape=oshape,
    )(rx, ry, rz)


def _lut_body(wd_ref, bd_ref, o0, o1, o2, o3):
    pid = pl.program_id(0)
    block = o0.shape[0]
    ridx = lax.broadcasted_iota(jnp.int32, (block, 1), 0) + pid * block
    d = jnp.maximum(ridx.astype(jnp.float32) * LSTEP, 3.16e-8)
    coefs = (
        lax.broadcasted_iota(jnp.int32, (1, N_RBF), 1).astype(jnp.float32) + 1.0
    ) * (math.pi / CUTOFF)
    rbf = jnp.sin(d * coefs) / d
    env = jnp.where(d < CUTOFF, 0.5 * (jnp.cos(d * (math.pi / CUTOFF)) + 1.0), 0.0)
    ws = (
        jnp.dot(rbf, wd_ref[...], preferred_element_type=jnp.float32) + bd_ref[...]
    ) * env
    pad = ws[:, 0:32]
    o0[...] = jnp.concatenate([ws[:, 0:96], pad], axis=1)
    o1[...] = jnp.concatenate([ws[:, 96:192], pad], axis=1)
    o2[...] = jnp.concatenate([ws[:, 192:288], pad], axis=1)
    o3[...] = jnp.concatenate([ws[:, 288:384], pad], axis=1)


def _lut(Wdp, bdp, block=2080):
    oshape = [jax.ShapeDtypeStruct((LUTR, 128), jnp.float32) for _ in range(NQ)]
    return pl.pallas_call(
        _lut_body,
        grid=(LUTR // block,),
        in_specs=[
            pl.BlockSpec((N_RBF, 3 * FEAT), lambda i: (0, 0)),
            pl.BlockSpec((1, 3 * FEAT), lambda i: (0, 0)),
        ],
        out_specs=[pl.BlockSpec((block, 128), lambda i: (i, 0)) for _ in range(NQ)],
        out_shape=oshape,
    )(Wdp, bdp.reshape(1, -1))


def _head_body(h_ref, w1_ref, b1_ref, w2_ref, b2_ref, out_ref):
    z = jnp.tanh(
        jnp.dot(h_ref[...], w1_ref[...], preferred_element_type=jnp.float32)
        + b1_ref[...]
    )
    out_ref[...] = (
        jnp.dot(z, w2_ref[...], preferred_element_type=jnp.float32) + b2_ref[...]
    )


def _head(h, W1, b1, W2, b2, block=1000):
    n = h.shape[0]
    return pl.pallas_call(
        _head_body,
        grid=(n // block,),
        in_specs=[
            pl.BlockSpec((block, FEAT), lambda i: (i, 0)),
            pl.BlockSpec((FEAT, FEAT), lambda i: (0, 0)),
            pl.BlockSpec((1, FEAT), lambda i: (0, 0)),
            pl.BlockSpec((FEAT, FEAT), lambda i: (0, 0)),
            pl.BlockSpec((1, FEAT), lambda i: (0, 0)),
        ],
        out_specs=pl.BlockSpec((block, FEAT), lambda i: (i, 0)),
        out_shape=jax.ShapeDtypeStruct((n, FEAT), jnp.float32),
    )(h, W1, b1.reshape(1, -1), W2, b2.reshape(1, -1))


_GATHER_DNUMS = lax.GatherDimensionNumbers(
    offset_dims=(), collapsed_slice_dims=(0,), start_index_map=(0,))


def _bcast(vec16, lane):
    idx = jnp.full((16, 1), lane, jnp.int32)
    return lax.gather(vec16, idx, _GATHER_DNUMS, slice_sizes=(1,),
                      mode=lax.GatherScatterMode.PROMISE_IN_BOUNDS)


def _sc_edge_kernel():
    mesh = plsc.VectorSubcoreMesh(
        core_axis_name="c", subcore_axis_name="s",
        num_cores=NCORES, num_subcores=NTILES,
    )
    oshape = tuple(
        jax.ShapeDtypeStruct((_NPAD, FEAT), jnp.float32) for _ in range(NQ)
    )

    @functools.partial(
        pl.kernel,
        out_type=oshape,
        mesh=mesh,
        scratch_types=[
            pltpu.VMEM_SHARED((_NPAD, FEAT), jnp.float32),
            pltpu.VMEM((4, KBLK), jnp.int32),
            pltpu.VMEM((4, KBLK), jnp.int32),
            pltpu.VMEM((4, KBLK), jnp.float32),
            pltpu.VMEM((4, KBLK), jnp.float32),
            pltpu.VMEM((KBLK, 128), jnp.float32),
            pltpu.VMEM((KBLK, 128), jnp.float32),
            pltpu.VMEM((KBLK, 128), jnp.float32),
            pltpu.VMEM((KBLK, 128), jnp.float32),
            pltpu.VMEM((KBLK, 128), jnp.float32),
            pltpu.VMEM((KBLK, 128), jnp.float32),
            pltpu.VMEM((KBLK, FEAT), jnp.float32),
            pltpu.SemaphoreType.DMA,
            pltpu.SemaphoreType.DMA,
            pltpu.SemaphoreType.DMA,
            pltpu.SemaphoreType.DMA,
        ],
    )
    def sc_edge(combi_h, combf_h, phi0, phi1, phi2, phi3, v0, v1, v2, v3,
                lut0, lut1, lut2, lut3, zeros_h,
                d0, d1, d2, d3,
                acc, combi0, combi1, combf0, combf1, phir0, phir1, vr0, vr1,
                lutr0, lutr1, out_v, sem_c0, sem_c1, sem_g0, sem_g1):
        c = lax.axis_index("c")
        s = lax.axis_index("s")
        combis = (combi0, combi1)
        combfs = (combf0, combf1)
        phirs = (phir0, phir1)
        vrs = (vr0, vr1)
        lutrs = (lutr0, lutr1)
        sem_cs = (sem_c0, sem_c1)
        sem_gs = (sem_g0, sem_g1)

        def run_pass(phi_h, v_h, lut_h, delta_h):
            pltpu.sync_copy(zeros_h, acc.at[pl.ds(s * RPT, RPT)])
            plsc.subcore_barrier()

            def issue_comb(b, p):
                bid = s * NBLK + b
                pltpu.async_copy(combi_h.at[bid], combis[p], sem_cs[p])
                pltpu.async_copy(combf_h.at[bid], combfs[p], sem_cs[p])

            def wait_comb(p):
                pltpu.make_async_copy(
                    combi_h.at[0], combis[p], sem_cs[p]).wait()
                pltpu.make_async_copy(
                    combf_h.at[0], combfs[p], sem_cs[p]).wait()

            def issue_gathers(p):
                pltpu.async_copy(phi_h.at[combis[p].at[1]], phirs[p], sem_gs[p])
                pltpu.async_copy(v_h.at[combis[p].at[1]], vrs[p], sem_gs[p])
                pltpu.async_copy(lut_h.at[combis[p].at[2]], lutrs[p], sem_gs[p])

            def wait_gathers(p):
                pltpu.make_async_copy(
                    phi_h.at[combis[p].at[1]], phirs[p], sem_gs[p]).wait()
                pltpu.make_async_copy(
                    v_h.at[combis[p].at[1]], vrs[p], sem_gs[p]).wait()
                pltpu.make_async_copy(
                    lut_h.at[combis[p].at[2]], lutrs[p], sem_gs[p]).wait()

            def compute(p):
                combp, phir, vr, lutr = combfs[p], phirs[p], vrs[p], lutrs[p]

                def group_body(g, carry):
                    e0w = (g // 2) * 16
                    w16 = combp[0, pl.ds(e0w, 16)]
                    ux16 = combp[1, pl.ds(e0w, 16)]
                    uy16 = combp[2, pl.ds(e0w, 16)]
                    uz16 = combp[3, pl.ds(e0w, 16)]
                    for l in range(8):
                        e = g * 8 + l
                        ll = e - e0w
                        bw = _bcast(w16, ll)
                        bux = _bcast(ux16, ll)
                        buy = _bcast(uy16, ll)
                        buz = _bcast(uz16, ll)
                        s00 = lutr[e, pl.ds(0, 16)] * bw
                        s01 = lutr[e, pl.ds(16, 16)] * bw
                        s10 = lutr[e, pl.ds(32, 16)] * bw
                        s11 = lutr[e, pl.ds(48, 16)] * bw
                        s20 = lutr[e, pl.ds(64, 16)] * bw
                        s21 = lutr[e, pl.ds(80, 16)] * bw
                        a00 = phir[e, pl.ds(0, 16)] * s00
                        a01 = phir[e, pl.ds(16, 16)] * s01
                        mh0 = phir[e, pl.ds(32, 16)] * s10
                        mh1 = phir[e, pl.ds(48, 16)] * s11
                        a20 = phir[e, pl.ds(64, 16)] * s20
                        a21 = phir[e, pl.ds(80, 16)] * s21
                        dvx0 = a20 * bux + a00 * vr[e, pl.ds(0, 16)]
                        dvx1 = a21 * bux + a01 * vr[e, pl.ds(16, 16)]
                        dvy0 = a20 * buy + a00 * vr[e, pl.ds(32, 16)]
                        dvy1 = a21 * buy + a01 * vr[e, pl.ds(48, 16)]
                        dvz0 = a20 * buz + a00 * vr[e, pl.ds(64, 16)]
                        dvz1 = a21 * buz + a01 * vr[e, pl.ds(80, 16)]
                        out_v[e, pl.ds(0, 16)] = dvx0
                        out_v[e, pl.ds(16, 16)] = dvx1
                        out_v[e, pl.ds(32, 16)] = dvy0
                        out_v[e, pl.ds(48, 16)] = dvy1
                        out_v[e, pl.ds(64, 16)] = dvz0
                        out_v[e, pl.ds(80, 16)] = dvz1
                        out_v[e, pl.ds(96, 16)] = mh0
                        out_v[e, pl.ds(112, 16)] = mh1
                    return carry

                lax.fori_loop(0, KBLK // 8, group_body, 0)

            def scatter(p):
                pltpu.sync_copy(out_v, acc.at[combis[p].at[0]], add=True)

            def do_block(b, p):
                wait_gathers(p)

                @pl.when(b + 1 < NBLK)
                def _():
                    wait_comb(1 - p)
                    issue_gathers(1 - p)

                compute(p)
                scatter(p)

                @pl.when(b + 2 < NBLK)
                def _():
                    issue_comb(b + 2, p)

            issue_comb(jnp.int32(0), 0)
            issue_comb(jnp.int32(1), 1)
            wait_comb(0)
            issue_gathers(0)

            def pair_body(t, carry):
                b = t * 2
                do_block(b, 0)
                do_block(b + 1, 1)
                return carry

            lax.fori_loop(0, NBLK // 2, pair_body, 0)

            plsc.subcore_barrier()
            pltpu.sync_copy(acc.at[pl.ds(s * RPT, RPT)],
                            delta_h.at[pl.ds(s * RPT, RPT)])
            plsc.subcore_barrier()

        @pl.when(c == 0)
        def _():
            run_pass(phi0, v0, lut0, d0)
            run_pass(phi1, v1, lut1, d1)

        @pl.when(c == 1)
        def _():
            run_pass(phi2, v2, lut2, d2)
            run_pass(phi3, v3, lut3, d3)

    return sc_edge


_SC_EDGE = _sc_edge_kernel()


def _perm_cols(w):
    shp = w.shape[:-1]
    return (
        w.reshape(shp + (3, NQ, QF))
        .swapaxes(-3, -2)
        .reshape(shp + (3 * FEAT,))
    )


def kernel(cg_z, cg_xyz, cg_nbr_list, emb, msg_W1, msg_b1, msg_W2, msg_b2,
           msg_Wd, msg_bd, mu_W1, mu_b1, mu_W2, mu_b2, sig_W1, sig_b1,
           sig_W2, sig_b2):
    E = cg_nbr_list.shape[0]
    N = cg_z.shape[0]
    E2 = 2 * E

    gtr_ij = (cg_nbr_list[:, 0] > cg_nbr_list[:, 1]).any()
    gtr_ji = (cg_nbr_list[:, 1] > cg_nbr_list[:, 0]).any()
    directed = jnp.logical_and(gtr_ij, gtr_ji)
    rev_w = jnp.where(directed, 0.0, 1.0).astype(jnp.float32)
    idx_i = jnp.concatenate([cg_nbr_list[:, 0], cg_nbr_list[:, 1]])
    idx_j = jnp.concatenate([cg_nbr_list[:, 1], cg_nbr_list[:, 0]])
    edge_w = jnp.concatenate(
        [jnp.ones((E,), jnp.float32), jnp.broadcast_to(rev_w, (E,))])

    r_ij = cg_xyz[idx_j] - cg_xyz[idx_i]
    rows = E2 // 128
    ux2, uy2, uz2, kk2 = _prep(
        r_ij[:, 0].reshape(rows, 128),
        r_ij[:, 1].reshape(rows, 128),
        r_ij[:, 2].reshape(rows, 128),
    )

    combi = jnp.stack([
        idx_i.astype(jnp.int32),
        idx_j.astype(jnp.int32),
        kk2.reshape(E2),
        jnp.zeros((E2,), jnp.int32),
    ])
    combi3 = (
        combi.reshape(4, NTILES, NBLK, KBLK)
        .transpose(1, 2, 0, 3)
        .reshape(NTILES * NBLK, 4, KBLK)
    )
    combf = jnp.stack([
        edge_w,
        ux2.reshape(E2),
        uy2.reshape(E2),
        uz2.reshape(E2),
    ])
    combf3 = (
        combf.reshape(4, NTILES, NBLK, KBLK)
        .transpose(1, 2, 0, 3)
        .reshape(NTILES * NBLK, 4, KBLK)
    )

    h = emb[cg_z]
    vq = [jnp.zeros((N, 128), jnp.float32) for _ in range(NQ)]
    zeros_stage = jnp.zeros((RPT, FEAT), jnp.float32)

    for i in range(N_CONV):
        phiq = _phi(h, msg_W1[i], msg_b1[i],
                    _perm_cols(msg_W2[i]), _perm_cols(msg_b2[i]))
        lutq = _lut(_perm_cols(msg_Wd[i]), _perm_cols(msg_bd[i]))
        deltas = _SC_EDGE(combi3, combf3, *phiq, *vq, *lutq, zeros_stage)
        dh = jnp.stack([d[:N, 96:128] for d in deltas], axis=1).reshape(N, FEAT)
        h = h + dh
        vq = [vq[k] + deltas[k][:N] for k in range(NQ)]

    H_mu = _head(h, mu_W1, mu_b1, mu_W2, mu_b2)
    H_sigma = _head(h, sig_W1, sig_b1, sig_W2, sig_b2)
    H_std = 1e-09 + jnp.exp(H_sigma / 2)
    return (H_mu, H_std)
